# baseline (device time: 39094 ns/iter reference)
import jax
import jax.numpy as jnp
from jax import lax
from jax.experimental import pallas as pl
from jax.experimental.pallas import tpu as pltpu

N_DEV = 32


def kernel(x, w_mat):
    m_per, k = x.shape
    n_per = w_mat.shape[1]
    m_tot = N_DEV * m_per

    def body(x_ref, w_ref, out_ref, xfull_ref, send_sems, recv_sems):
        my = lax.axis_index("i")

        my_rows = pl.ds(my * m_per, m_per)
        xfull_ref[my_rows, :] = x_ref[...].astype(jnp.bfloat16)

        rdmas = []
        for off in range(1, N_DEV):
            tgt = lax.rem(my + off, N_DEV)
            rdma = pltpu.make_async_remote_copy(
                src_ref=xfull_ref.at[my_rows, :],
                dst_ref=xfull_ref.at[my_rows, :],
                send_sem=send_sems.at[off - 1],
                recv_sem=recv_sems.at[off - 1],
                device_id=(tgt,),
                device_id_type=pl.DeviceIdType.MESH,
            )
            rdma.start()
            rdmas.append(rdma)

        for rdma in rdmas:
            rdma.wait_recv()
        for rdma in rdmas:
            rdma.wait_send()

        acc = jnp.dot(
            xfull_ref[...],
            w_ref[...].astype(jnp.bfloat16),
            preferred_element_type=jnp.float32,
        )
        out_ref[...] = jnp.maximum(acc, 0.0)

    return pl.pallas_call(
        body,
        out_shape=jax.ShapeDtypeStruct((m_tot, n_per), jnp.float32),
        in_specs=[
            pl.BlockSpec(memory_space=pltpu.VMEM),
            pl.BlockSpec(memory_space=pltpu.VMEM),
        ],
        out_specs=pl.BlockSpec(memory_space=pltpu.VMEM),
        scratch_shapes=[
            pltpu.VMEM((m_tot, k), jnp.bfloat16),
            pltpu.SemaphoreType.DMA((N_DEV - 1,)),
            pltpu.SemaphoreType.DMA((N_DEV - 1,)),
        ],
    )(x, w_mat)


# device time: 31271 ns/iter; 1.2502x vs baseline; 1.2502x over previous
import jax
import jax.numpy as jnp
from jax import lax
from jax.experimental import pallas as pl
from jax.experimental.pallas import tpu as pltpu

N_DEV = 32
N_Z = 4
N_P = 8


def _succ_p(p):
    return jnp.where(p == 0, 1, jnp.where(p == 1, 2, jnp.where(
        p == 2, 5, jnp.where(p == 3, 0, jnp.where(p == 4, 3, jnp.where(
            p == 5, 6, jnp.where(p == 6, 7, 4)))))))


def _pred_p(p):
    return jnp.where(p == 0, 3, jnp.where(p == 1, 0, jnp.where(
        p == 2, 1, jnp.where(p == 3, 4, jnp.where(p == 4, 7, jnp.where(
            p == 5, 2, jnp.where(p == 6, 5, 6)))))))


def kernel(x, w_mat):
    m_per, k = x.shape
    n_per = w_mat.shape[1]
    m_tot = N_DEV * m_per

    def body(x_ref, w_ref, out_ref, xfull_ref,
             sz_send, sz_recv, cw_send, cw_recv, ccw_send, ccw_recv):
        my = lax.axis_index("i")
        my_z = lax.div(my, N_P)
        my_p = lax.rem(my, N_P)
        succ_id = N_P * my_z + _succ_p(my_p)
        pred_id = N_P * my_z + _pred_p(my_p)

        def chunk_rows(dev_id):
            return pl.ds(dev_id * m_per, m_per)

        barrier_sem = pltpu.get_barrier_semaphore()
        peers = [succ_id, pred_id]
        for dz in range(1, N_Z):
            peers.append(N_P * lax.rem(my_z + dz, N_Z) + my_p)
        for peer in peers:
            pl.semaphore_signal(
                barrier_sem, inc=1,
                device_id=(peer,), device_id_type=pl.DeviceIdType.MESH,
            )
        pl.semaphore_wait(barrier_sem, len(peers))

        xfull_ref[chunk_rows(my), :] = x_ref[...].astype(jnp.bfloat16)

        z_rdmas = []
        for dz in range(1, N_Z):
            tgt = N_P * lax.rem(my_z + dz, N_Z) + my_p
            rdma = pltpu.make_async_remote_copy(
                src_ref=xfull_ref.at[chunk_rows(my), :],
                dst_ref=xfull_ref.at[chunk_rows(my), :],
                send_sem=sz_send.at[dz - 1],
                recv_sem=sz_recv.at[dz - 1],
                device_id=(tgt,),
                device_id_type=pl.DeviceIdType.MESH,
            )
            rdma.start()
            z_rdmas.append(rdma)
        for rdma in z_rdmas:
            rdma.wait_recv()

        o_cw = [my_p]
        o_ccw = [my_p]
        for _ in range(N_P // 2 - 1):
            o_cw.append(_pred_p(o_cw[-1]))
            o_ccw.append(_succ_p(o_ccw[-1]))

        def start_block(origin_p, tgt, send_sems, recv_sems, t):
            rdmas = []
            for zz in range(N_Z):
                rows = chunk_rows(N_P * zz + origin_p)
                rdma = pltpu.make_async_remote_copy(
                    src_ref=xfull_ref.at[rows, :],
                    dst_ref=xfull_ref.at[rows, :],
                    send_sem=send_sems.at[t, zz],
                    recv_sem=recv_sems.at[t, zz],
                    device_id=(tgt,),
                    device_id_type=pl.DeviceIdType.MESH,
                )
                rdma.start()
                rdmas.append(rdma)
            return rdmas

        cw_rdmas, ccw_rdmas = [], []
        for t in range(4):
            if t > 0:
                for rdma in cw_rdmas[t - 1]:
                    rdma.wait_recv()
            cw_rdmas.append(start_block(o_cw[t], succ_id, cw_send, cw_recv, t))
            if t < 3:
                if t > 0:
                    for rdma in ccw_rdmas[t - 1]:
                        rdma.wait_recv()
                ccw_rdmas.append(
                    start_block(o_ccw[t], pred_id, ccw_send, ccw_recv, t))
        for rdma in cw_rdmas[3]:
            rdma.wait_recv()
        for rdma in ccw_rdmas[2]:
            rdma.wait_recv()

        for group in [z_rdmas] + cw_rdmas + ccw_rdmas:
            for rdma in group:
                rdma.wait_send()

        acc = jnp.dot(
            xfull_ref[...],
            w_ref[...].astype(jnp.bfloat16),
            preferred_element_type=jnp.float32,
        )
        out_ref[...] = jnp.maximum(acc, 0.0)

    return pl.pallas_call(
        body,
        out_shape=jax.ShapeDtypeStruct((m_tot, n_per), jnp.float32),
        in_specs=[
            pl.BlockSpec(memory_space=pltpu.VMEM),
            pl.BlockSpec(memory_space=pltpu.VMEM),
        ],
        out_specs=pl.BlockSpec(memory_space=pltpu.VMEM),
        scratch_shapes=[
            pltpu.VMEM((m_tot, k), jnp.bfloat16),
            pltpu.SemaphoreType.DMA((N_Z - 1,)),
            pltpu.SemaphoreType.DMA((N_Z - 1,)),
            pltpu.SemaphoreType.DMA((4, N_Z)),
            pltpu.SemaphoreType.DMA((4, N_Z)),
            pltpu.SemaphoreType.DMA((3, N_Z)),
            pltpu.SemaphoreType.DMA((3, N_Z)),
        ],
        compiler_params=pltpu.CompilerParams(collective_id=0),
    )(x, w_mat)


# device time: 24046 ns/iter; 1.6258x vs baseline; 1.3005x over previous
import jax
import jax.numpy as jnp
from jax import lax
from jax.experimental import pallas as pl
from jax.experimental.pallas import tpu as pltpu

N_DEV = 32
N_Z = 4
N_P = 8


def _succ_p(p):
    return jnp.where(p == 0, 1, jnp.where(p == 1, 2, jnp.where(
        p == 2, 5, jnp.where(p == 3, 0, jnp.where(p == 4, 3, jnp.where(
            p == 5, 6, jnp.where(p == 6, 7, 4)))))))


def _pred_p(p):
    return jnp.where(p == 0, 3, jnp.where(p == 1, 0, jnp.where(
        p == 2, 1, jnp.where(p == 3, 4, jnp.where(p == 4, 7, jnp.where(
            p == 5, 2, jnp.where(p == 6, 5, 6)))))))


def kernel(x, w_mat):
    m_per, k = x.shape
    n_per = w_mat.shape[1]
    m_tot = N_DEV * m_per

    def body(x_ref, w_ref, out_ref, xfull_ref,
             sz_send, sz_recv, cw_send, cw_recv, ccw_send, ccw_recv):
        my = lax.axis_index("i")
        my_z = lax.div(my, N_P)
        my_p = lax.rem(my, N_P)
        succ_id = N_P * my_z + _succ_p(my_p)
        pred_id = N_P * my_z + _pred_p(my_p)

        def chunk_rows(dev_id):
            return pl.ds(dev_id * m_per, m_per)

        barrier_sem = pltpu.get_barrier_semaphore()
        peers = [succ_id, pred_id]
        for dz in range(1, N_Z):
            peers.append(N_P * lax.rem(my_z + dz, N_Z) + my_p)
        for peer in peers:
            pl.semaphore_signal(
                barrier_sem, inc=1,
                device_id=(peer,), device_id_type=pl.DeviceIdType.MESH,
            )
        pl.semaphore_wait(barrier_sem, len(peers))

        xfull_ref[chunk_rows(my), :] = x_ref[...].astype(jnp.bfloat16)

        z_rdmas = []
        for dz in range(1, N_Z):
            tgt = N_P * lax.rem(my_z + dz, N_Z) + my_p
            rdma = pltpu.make_async_remote_copy(
                src_ref=xfull_ref.at[chunk_rows(my), :],
                dst_ref=xfull_ref.at[chunk_rows(my), :],
                send_sem=sz_send.at[dz - 1],
                recv_sem=sz_recv.at[dz - 1],
                device_id=(tgt,),
                device_id_type=pl.DeviceIdType.MESH,
            )
            rdma.start()
            z_rdmas.append(rdma)

        zz_of = [lax.rem(my_z - dz + N_Z, N_Z) for dz in range(N_Z)]
        o_cw = [my_p]
        o_ccw = [my_p]
        for _ in range(N_P // 2 - 1):
            o_cw.append(_pred_p(o_cw[-1]))
            o_ccw.append(_succ_p(o_ccw[-1]))

        def start_one(dz, t, origin_p, tgt, send_sems, recv_sems):
            rows = chunk_rows(N_P * zz_of[dz] + origin_p)
            rdma = pltpu.make_async_remote_copy(
                src_ref=xfull_ref.at[rows, :],
                dst_ref=xfull_ref.at[rows, :],
                send_sem=send_sems.at[dz, t],
                recv_sem=recv_sems.at[dz, t],
                device_id=(tgt,),
                device_id_type=pl.DeviceIdType.MESH,
            )
            rdma.start()
            return rdma

        cw_rdmas = [[] for _ in range(N_Z)]
        ccw_rdmas = [[] for _ in range(N_Z)]
        for dz in range(N_Z):
            if dz > 0:
                z_rdmas[dz - 1].wait_recv()
            cw_rdmas[dz].append(
                start_one(dz, 0, o_cw[0], succ_id, cw_send, cw_recv))
            ccw_rdmas[dz].append(
                start_one(dz, 0, o_ccw[0], pred_id, ccw_send, ccw_recv))
        for t in range(1, 4):
            for dz in range(N_Z):
                cw_rdmas[dz][t - 1].wait_recv()
                cw_rdmas[dz].append(
                    start_one(dz, t, o_cw[t], succ_id, cw_send, cw_recv))
                if t < 3:
                    ccw_rdmas[dz][t - 1].wait_recv()
                    ccw_rdmas[dz].append(
                        start_one(dz, t, o_ccw[t], pred_id, ccw_send, ccw_recv))
        for dz in range(N_Z):
            cw_rdmas[dz][3].wait_recv()
            ccw_rdmas[dz][2].wait_recv()

        for group in [z_rdmas] + cw_rdmas + ccw_rdmas:
            for rdma in group:
                rdma.wait_send()

        acc = jnp.dot(
            xfull_ref[...],
            w_ref[...].astype(jnp.bfloat16),
            preferred_element_type=jnp.float32,
        )
        out_ref[...] = jnp.maximum(acc, 0.0)

    return pl.pallas_call(
        body,
        out_shape=jax.ShapeDtypeStruct((m_tot, n_per), jnp.float32),
        in_specs=[
            pl.BlockSpec(memory_space=pltpu.VMEM),
            pl.BlockSpec(memory_space=pltpu.VMEM),
        ],
        out_specs=pl.BlockSpec(memory_space=pltpu.VMEM),
        scratch_shapes=[
            pltpu.VMEM((m_tot, k), jnp.bfloat16),
            pltpu.SemaphoreType.DMA((N_Z - 1,)),
            pltpu.SemaphoreType.DMA((N_Z - 1,)),
            pltpu.SemaphoreType.DMA((N_Z, 4)),
            pltpu.SemaphoreType.DMA((N_Z, 4)),
            pltpu.SemaphoreType.DMA((N_Z, 3)),
            pltpu.SemaphoreType.DMA((N_Z, 3)),
        ],
        compiler_params=pltpu.CompilerParams(collective_id=0),
    )(x, w_mat)


# device time: 23549 ns/iter; 1.6601x vs baseline; 1.0211x over previous
import jax
import jax.numpy as jnp
from jax import lax
from jax.experimental import pallas as pl
from jax.experimental.pallas import tpu as pltpu

N_DEV = 32
N_Z = 4
N_P = 8


def _succ_p(p):
    return jnp.where(p == 0, 1, jnp.where(p == 1, 2, jnp.where(
        p == 2, 5, jnp.where(p == 3, 0, jnp.where(p == 4, 3, jnp.where(
            p == 5, 6, jnp.where(p == 6, 7, 4)))))))


def _pred_p(p):
    return jnp.where(p == 0, 3, jnp.where(p == 1, 0, jnp.where(
        p == 2, 1, jnp.where(p == 3, 4, jnp.where(p == 4, 7, jnp.where(
            p == 5, 2, jnp.where(p == 6, 5, 6)))))))


def kernel(x, w_mat):
    m_per, k = x.shape
    n_per = w_mat.shape[1]
    m_tot = N_DEV * m_per

    def body(x_ref, w_ref, out_ref, xfull_ref,
             sz_send, sz_recv, cw_send, cw_recv, ccw_send, ccw_recv):
        my = lax.axis_index("i")
        my_z = lax.div(my, N_P)
        my_p = lax.rem(my, N_P)
        succ_id = N_P * my_z + _succ_p(my_p)
        pred_id = N_P * my_z + _pred_p(my_p)

        def chunk_rows(dev_id):
            return pl.ds(dev_id * m_per, m_per)

        barrier_sem = pltpu.get_barrier_semaphore()
        peers = [succ_id, pred_id]
        for dz in range(1, N_Z):
            peers.append(N_P * lax.rem(my_z + dz, N_Z) + my_p)
        for peer in peers:
            pl.semaphore_signal(
                barrier_sem, inc=1,
                device_id=(peer,), device_id_type=pl.DeviceIdType.MESH,
            )
        pl.semaphore_wait(barrier_sem, len(peers))

        xfull_ref[chunk_rows(my), :] = x_ref[...].astype(jnp.bfloat16)

        z_rdmas = []
        for dz in range(1, N_Z):
            tgt = N_P * lax.rem(my_z + dz, N_Z) + my_p
            rdma = pltpu.make_async_remote_copy(
                src_ref=xfull_ref.at[chunk_rows(my), :],
                dst_ref=xfull_ref.at[chunk_rows(my), :],
                send_sem=sz_send.at[dz - 1],
                recv_sem=sz_recv.at[dz - 1],
                device_id=(tgt,),
                device_id_type=pl.DeviceIdType.MESH,
            )
            rdma.start()
            z_rdmas.append(rdma)

        zz_of = [lax.rem(my_z - dz + N_Z, N_Z) for dz in range(N_Z)]
        o_cw = [my_p]
        o_ccw = [my_p]
        for _ in range(N_P // 2 - 1):
            o_cw.append(_pred_p(o_cw[-1]))
            o_ccw.append(_succ_p(o_ccw[-1]))

        def start_one(dz, t, origin_p, tgt, send_sems, recv_sems):
            rows = chunk_rows(N_P * zz_of[dz] + origin_p)
            rdma = pltpu.make_async_remote_copy(
                src_ref=xfull_ref.at[rows, :],
                dst_ref=xfull_ref.at[rows, :],
                send_sem=send_sems.at[dz, t],
                recv_sem=recv_sems.at[dz, t],
                device_id=(tgt,),
                device_id_type=pl.DeviceIdType.MESH,
            )
            rdma.start()
            return rdma

        cw_rdmas = [[] for _ in range(N_Z)]
        ccw_rdmas = [[] for _ in range(N_Z)]
        for dz in range(N_Z):
            if dz > 0:
                z_rdmas[dz - 1].wait_recv()
            cw_rdmas[dz].append(
                start_one(dz, 0, o_cw[0], succ_id, cw_send, cw_recv))
            ccw_rdmas[dz].append(
                start_one(dz, 0, o_ccw[0], pred_id, ccw_send, ccw_recv))
        for t in range(1, 4):
            for dz in range(N_Z):
                cw_rdmas[dz][t - 1].wait_recv()
                cw_rdmas[dz].append(
                    start_one(dz, t, o_cw[t], succ_id, cw_send, cw_recv))
                if t < 3:
                    ccw_rdmas[dz][t - 1].wait_recv()
                    ccw_rdmas[dz].append(
                        start_one(dz, t, o_ccw[t], pred_id, ccw_send, ccw_recv))
        w_bf = w_ref[...].astype(jnp.bfloat16)
        for dz in range(N_Z):
            cw_rdmas[dz][3].wait_recv()
            ccw_rdmas[dz][2].wait_recv()
            layer_rows = pl.ds(N_P * zz_of[dz] * m_per, N_P * m_per)
            acc = jnp.dot(
                xfull_ref[layer_rows, :], w_bf,
                preferred_element_type=jnp.float32,
            )
            out_ref[layer_rows, :] = jnp.maximum(acc, 0.0)

        for group in [z_rdmas] + cw_rdmas + ccw_rdmas:
            for rdma in group:
                rdma.wait_send()

    return pl.pallas_call(
        body,
        out_shape=jax.ShapeDtypeStruct((m_tot, n_per), jnp.float32),
        in_specs=[
            pl.BlockSpec(memory_space=pltpu.VMEM),
            pl.BlockSpec(memory_space=pltpu.VMEM),
        ],
        out_specs=pl.BlockSpec(memory_space=pltpu.VMEM),
        scratch_shapes=[
            pltpu.VMEM((m_tot, k), jnp.bfloat16),
            pltpu.SemaphoreType.DMA((N_Z - 1,)),
            pltpu.SemaphoreType.DMA((N_Z - 1,)),
            pltpu.SemaphoreType.DMA((N_Z, 4)),
            pltpu.SemaphoreType.DMA((N_Z, 4)),
            pltpu.SemaphoreType.DMA((N_Z, 3)),
            pltpu.SemaphoreType.DMA((N_Z, 3)),
        ],
        compiler_params=pltpu.CompilerParams(collective_id=0),
    )(x, w_mat)
